# Initial kernel scaffold; baseline (speedup 1.0000x reference)
import jax
import jax.numpy as jnp
from jax.experimental import pallas as pl


def kernel(x, pos, edge_index, leaf_ids, readout_ids, emb, norm_g, norm_b, Wq, Wk, Wv, Wo, W1, b1, W2, b2, ln1_g, ln1_b, ln2_g, ln2_b, W_gen, b_gen):
    raise NotImplementedError("write your pallas kernel here")


# speedup vs baseline: 13.5245x; 13.5245x over previous
import functools

import jax
import jax.numpy as jnp
import numpy as np
from jax import lax
from jax.experimental import pallas as pl
from jax.experimental.pallas import tpu as pltpu
from jax.experimental.pallas import tpu_sc as plsc

N = 10000
E = 160000
NL = 5000
NR = 100
V = 10000
D = 256
DFF = 1024
H = 8
L = 3
NC = 128
DH = D // H

_NSC = 2
_NT = 16
_CH = 80
_EPT = E // _NT
_NCHUNK = _EPT // _CH
_HHALF = D // _NSC
_HPS = H // _NSC
_ND = 640
_RPT = N // _NT
_NLP = 5120
_INV = 1.0 / np.sqrt(DH)

_mesh = plsc.VectorSubcoreMesh(core_axis_name="c", subcore_axis_name="s")


@functools.partial(
    pl.kernel,
    mesh=_mesh,
    out_type=jax.ShapeDtypeStruct((_NLP, D), jnp.float32),
    scratch_types=[
        pltpu.VMEM((_CH,), jnp.int32),
        pltpu.VMEM((_CH, D), jnp.float32),
        pltpu.SemaphoreType.DMA,
    ],
)
def _embed_gather(emb_hbm, x_hbm, out_hbm, ib, rows, sem):
    c = lax.axis_index("c")
    s = lax.axis_index("s")
    w = s * _NSC + c
    for kk in range(_NLP // (32 * _CH)):
        base = pl.multiple_of(w * (_NLP // 32) + kk * _CH, 8)
        pltpu.sync_copy(x_hbm.at[pl.ds(base, _CH)], ib)
        pltpu.async_copy(emb_hbm.at[ib], rows, sem).wait()
        pltpu.sync_copy(rows, out_hbm.at[pl.ds(base, _CH)])


@functools.partial(
    pl.kernel,
    mesh=_mesh,
    out_type=jax.ShapeDtypeStruct((_NSC * E, _HHALF), jnp.float32),
    scratch_types=[
        pltpu.VMEM((_CH,), jnp.int32),
        pltpu.VMEM((_CH,), jnp.int32),
        pltpu.VMEM((_CH, _HHALF), jnp.float32),
        pltpu.VMEM((_CH, _HHALF), jnp.float32),
        pltpu.SemaphoreType.DMA,
    ],
)
def _edgeprod_sc(qc, kc, srcs, dsts, out, sbuf, dbuf, qd, ks, sem):
    c = lax.axis_index("c")
    s = lax.axis_index("s")
    cn = c * N

    def _chunk(ci, carry):
        start = pl.multiple_of(s * _EPT + ci * _CH, 8)
        pltpu.sync_copy(srcs.at[pl.ds(start, _CH)], sbuf)
        pltpu.sync_copy(dsts.at[pl.ds(start, _CH)], dbuf)
        for i in range(_CH // 16):
            sl = pl.ds(i * 16, 16)
            sbuf[sl] = sbuf[sl] + cn
            dbuf[sl] = dbuf[sl] + cn
        cp1 = pltpu.async_copy(qc.at[dbuf], qd, sem)
        cp2 = pltpu.async_copy(kc.at[sbuf], ks, sem)
        cp1.wait()
        cp2.wait()

        def _edge(e, carry2):
            for half in range(_HHALF // 16):
                sl = pl.ds(half * 16, 16)
                qd[e, sl] = qd[e, sl] * ks[e, sl]
            return carry2

        lax.fori_loop(0, _CH, _edge, 0)
        pltpu.sync_copy(qd, out.at[pl.ds(c * E + start, _CH)])
        return carry

    lax.fori_loop(0, _NCHUNK, _chunk, 0)


_BE = 2000


def _score_body(pe_ref, dst_ref, aw_ref, den_ref):
    pfull = jnp.concatenate([pe_ref[0], pe_ref[1]], axis=1)
    cc0 = lax.broadcasted_iota(jnp.int32, (D, H), 0)
    hh = lax.broadcasted_iota(jnp.int32, (D, H), 1)
    sel = ((cc0 // DH) == hh).astype(jnp.float32)
    a = jnp.exp(jnp.minimum(
        jnp.dot(pfull, sel, preferred_element_type=jnp.float32) * _INV,
        60.0))
    jj = lax.broadcasted_iota(jnp.int32, (_HPS, _HHALF), 0)
    col = lax.broadcasted_iota(jnp.int32, (_HPS, _HHALF), 1)
    rep = ((col // DH) == jj).astype(jnp.float32)
    aw_ref[0] = jnp.dot(a[:, :_HPS], rep, preferred_element_type=jnp.float32)
    aw_ref[1] = jnp.dot(a[:, _HPS:], rep, preferred_element_type=jnp.float32)
    a_rep = jnp.concatenate([a] * 16, axis=1)
    lanes = lax.broadcasted_iota(jnp.int32, (_BE, _HHALF), 1)
    m = (lanes // H) == (dst_ref[...] % 16)
    den_ref[...] = a_rep * m.astype(jnp.float32)


def _score_tc(pe, dst2):
    grid = (E // _BE,)
    return pl.pallas_call(
        _score_body,
        grid=grid,
        in_specs=[
            pl.BlockSpec((_NSC, _BE, _HHALF), lambda i: (0, i, 0)),
            pl.BlockSpec((_BE, 1), lambda i: (i, 0)),
        ],
        out_specs=(
            pl.BlockSpec((_NSC, _BE, _HHALF), lambda i: (0, i, 0)),
            pl.BlockSpec((_BE, _HHALF), lambda i: (i, 0)),
        ),
        out_shape=(
            jax.ShapeDtypeStruct((_NSC, E, _HHALF), jnp.float32),
            jax.ShapeDtypeStruct((E, _HHALF), jnp.float32),
        ),
    )(pe, dst2)


@functools.partial(
    pl.kernel,
    mesh=_mesh,
    out_type=(
        jax.ShapeDtypeStruct((_NSC * N, _HHALF), jnp.float32),
        jax.ShapeDtypeStruct((_NSC * _ND, _HHALF), jnp.float32),
    ),
    scratch_types=[
        pltpu.VMEM((_CH,), jnp.int32),
        pltpu.VMEM((_CH,), jnp.int32),
        pltpu.VMEM((_CH,), jnp.int32),
        pltpu.VMEM((_CH, _HHALF), jnp.float32),
        pltpu.VMEM((_CH, _HHALF), jnp.float32),
        pltpu.VMEM((_CH, _HHALF), jnp.float32),
        pltpu.VMEM((_CH, _HHALF), jnp.float32),
        pltpu.VMEM_SHARED((N, _HHALF), jnp.float32),
        pltpu.VMEM_SHARED((_ND, _HHALF), jnp.float32),
        pltpu.SemaphoreType.DMA,
    ],
)
def _attn_sc(vc, aw, den, srcs, dsts, out, dout, sbuf, dbuf, dbuf16,
             vs, awb, denb, contrib, shared, sharedd, sem):
    c = lax.axis_index("c")
    s = lax.axis_index("s")
    cn = c * N
    zero16 = jnp.zeros((16,), jnp.float32)

    def _zrow(r, carry):
        for c16 in range(_HHALF // 16):
            contrib[r, pl.ds(c16 * 16, 16)] = zero16
        return carry

    lax.fori_loop(0, _CH, _zrow, 0)
    nchn = N // _CH

    def _zchunk(k, carry):
        j = jnp.minimum(s + k * _NT, nchn - 1)
        base = pl.multiple_of(j * _CH, 8)
        pltpu.sync_copy(contrib, shared.at[pl.ds(base, _CH)])
        return carry

    lax.fori_loop(0, (nchn + _NT - 1) // _NT, _zchunk, 0)
    jd = jnp.minimum(s, _ND // _CH - 1)
    based = pl.multiple_of(jd * _CH, 8)
    pltpu.sync_copy(contrib, sharedd.at[pl.ds(based, _CH)])
    plsc.subcore_barrier()

    def _chunk(ci, carry):
        start = pl.multiple_of(s * _EPT + ci * _CH, 8)
        pltpu.sync_copy(srcs.at[pl.ds(start, _CH)], sbuf)
        pltpu.sync_copy(dsts.at[pl.ds(start, _CH)], dbuf)
        for i in range(_CH // 16):
            sl = pl.ds(i * 16, 16)
            dbuf16[sl] = lax.shift_right_logical(dbuf[sl], 4)
            sbuf[sl] = sbuf[sl] + cn
        cp1 = pltpu.async_copy(vc.at[sbuf], vs, sem)
        cp2 = pltpu.async_copy(aw.at[pl.ds(c * E + start, _CH)], awb, sem)
        cp3 = pltpu.async_copy(den.at[pl.ds(start, _CH)], denb, sem)
        cp1.wait()
        cp2.wait()
        cp3.wait()

        def _edge(e, carry2):
            for half in range(_HHALF // 16):
                sl = pl.ds(half * 16, 16)
                contrib[e, sl] = awb[e, sl] * vs[e, sl]
            return carry2

        lax.fori_loop(0, _CH, _edge, 0)
        pltpu.sync_copy(contrib, shared.at[dbuf], add=True)
        pltpu.sync_copy(denb, sharedd.at[dbuf16], add=True)
        return carry

    lax.fori_loop(0, _NCHUNK, _chunk, 0)
    plsc.subcore_barrier()

    def _wchunk(k, carry):
        j = jnp.minimum(s + k * _NT, nchn - 1)
        base = pl.multiple_of(j * _CH, 8)
        pltpu.sync_copy(shared.at[pl.ds(base, _CH)],
                        out.at[pl.ds(cn + base, _CH)])
        return carry

    lax.fori_loop(0, (nchn + _NT - 1) // _NT, _wchunk, 0)
    pltpu.sync_copy(sharedd.at[pl.ds(based, _CH)],
                    dout.at[pl.ds(c * _ND + based, _CH)])


_BN = 1000


def _ln(x, g, b):
    m = jnp.mean(x, axis=-1, keepdims=True)
    v = jnp.mean((x - m) ** 2, axis=-1, keepdims=True)
    return (x - m) / jnp.sqrt(v + 1e-5) * g + b


def _pe_ln_body(her_ref, posf_ref, g_ref, b_ref, out_ref):
    her = her_ref[...]
    posf = posf_ref[...]
    ci = lax.broadcasted_iota(jnp.int32, (her.shape[0], D), 1)
    i_f = (ci // 2).astype(jnp.float32)
    freq = jnp.exp((-np.log(10000.0) * 2.0 / D) * i_f)
    ang = posf * freq
    pe = jnp.where((ci % 2) == 0, jnp.sin(ang), jnp.cos(ang))
    out_ref[...] = _ln(her + pe, g_ref[...], b_ref[...])


def _scatter_body(he_ref, ids_ref, out_ref):
    out_ref[...] = jnp.zeros((N, D), jnp.float32)

    def body(i, carry):
        idv = ids_ref[i]
        out_ref[pl.ds(idv, 1), :] = he_ref[pl.ds(i, 1), :]
        return carry

    lax.fori_loop(0, NL, body, 0)


def _qkv_body(h_ref, wq_ref, wk_ref, wv_ref, q_ref, k_ref, v_ref):
    h = h_ref[...]
    for w_ref, o_ref in ((wq_ref, q_ref), (wk_ref, k_ref), (wv_ref, v_ref)):
        o = jnp.dot(h, w_ref[...], preferred_element_type=jnp.float32)
        o_ref[0] = o[:, :_HHALF]
        o_ref[1] = o[:, _HHALF:]


def _post_body(h_ref, sc_ref, den_ref, wo_ref, w1_ref, b1_ref, w2_ref,
               b2_ref, g1_ref, bb1_ref, g2_ref, bb2_ref, out_ref):
    h = h_ref[...]
    num = jnp.concatenate([sc_ref[0], sc_ref[1]], axis=1)
    den8 = den_ref[...] + 1e-9
    hh = lax.broadcasted_iota(jnp.int32, (H, D), 0)
    cc = lax.broadcasted_iota(jnp.int32, (H, D), 1)
    sel = ((cc // DH) == hh).astype(jnp.float32)
    den = jnp.dot(den8, sel, preferred_element_type=jnp.float32)
    agg = num / den
    o = jnp.dot(agg, wo_ref[...], preferred_element_type=jnp.float32)
    h1 = _ln(h + o, g1_ref[...], bb1_ref[...])
    f = jnp.maximum(
        jnp.dot(h1, w1_ref[...], preferred_element_type=jnp.float32)
        + b1_ref[...], 0.0)
    f2 = jnp.dot(f, w2_ref[...], preferred_element_type=jnp.float32) \
        + b2_ref[...]
    out_ref[...] = _ln(h1 + f2, g2_ref[...], bb2_ref[...])


def _readout_body(h_ref, ids_ref, wg_ref, bg_ref, out_ref, rows_ref):
    def body(j, carry):
        rid = ids_ref[j]
        rows_ref[pl.ds(j, 1), :] = h_ref[pl.ds(rid, 1), :]
        return carry

    lax.fori_loop(0, NR, body, 0)
    xs = rows_ref[...]
    z = jnp.dot(xs, wg_ref[...], preferred_element_type=jnp.float32) \
        + bg_ref[...]
    m = jnp.max(z, axis=-1, keepdims=True)
    out_ref[...] = z - m - jnp.log(
        jnp.sum(jnp.exp(z - m), axis=-1, keepdims=True))


def _full_spec(shape):
    return pl.BlockSpec(shape, lambda *_: (0,) * len(shape))


def _pe_ln(he_raw, posf, norm_g, norm_b):
    grid = (NL // _BN,)
    return pl.pallas_call(
        _pe_ln_body,
        grid=grid,
        in_specs=[
            pl.BlockSpec((_BN, D), lambda i: (i, 0)),
            pl.BlockSpec((_BN, 1), lambda i: (i, 0)),
            _full_spec((1, D)),
            _full_spec((1, D)),
        ],
        out_specs=pl.BlockSpec((_BN, D), lambda i: (i, 0)),
        out_shape=jax.ShapeDtypeStruct((NL, D), jnp.float32),
    )(he_raw, posf, norm_g.reshape(1, D), norm_b.reshape(1, D))


def _scatter_leaves(he, leaf_ids):
    return pl.pallas_call(
        _scatter_body,
        in_specs=[
            pl.BlockSpec(memory_space=pltpu.MemorySpace.VMEM),
            pl.BlockSpec(memory_space=pltpu.MemorySpace.SMEM),
        ],
        out_specs=pl.BlockSpec(memory_space=pltpu.MemorySpace.VMEM),
        out_shape=jax.ShapeDtypeStruct((N, D), jnp.float32),
        compiler_params=pltpu.CompilerParams(
            vmem_limit_bytes=100 * 2**20),
    )(he, leaf_ids)


def _qkv(h, wq, wk, wv):
    grid = (N // _BN,)
    o_spec = pl.BlockSpec((_NSC, _BN, _HHALF), lambda i: (0, i, 0))
    o_shape = jax.ShapeDtypeStruct((_NSC, N, _HHALF), jnp.float32)
    return pl.pallas_call(
        _qkv_body,
        grid=grid,
        in_specs=[
            pl.BlockSpec((_BN, D), lambda i: (i, 0)),
            _full_spec((D, D)),
            _full_spec((D, D)),
            _full_spec((D, D)),
        ],
        out_specs=(o_spec, o_spec, o_spec),
        out_shape=(o_shape, o_shape, o_shape),
    )(h, wq, wk, wv)


def _post(h, sc_out, den8, wo, w1, b1, w2, b2, g1, bb1, g2, bb2):
    grid = (N // _BN,)
    return pl.pallas_call(
        _post_body,
        grid=grid,
        in_specs=[
            pl.BlockSpec((_BN, D), lambda i: (i, 0)),
            pl.BlockSpec((_NSC, _BN, _HHALF), lambda i: (0, i, 0)),
            pl.BlockSpec((_BN, H), lambda i: (i, 0)),
            _full_spec((D, D)),
            _full_spec((D, DFF)),
            _full_spec((1, DFF)),
            _full_spec((DFF, D)),
            _full_spec((1, D)),
            _full_spec((1, D)),
            _full_spec((1, D)),
            _full_spec((1, D)),
            _full_spec((1, D)),
        ],
        out_specs=pl.BlockSpec((_BN, D), lambda i: (i, 0)),
        out_shape=jax.ShapeDtypeStruct((N, D), jnp.float32),
    )(h, sc_out, den8, wo, w1, b1.reshape(1, DFF), w2, b2.reshape(1, D),
      g1.reshape(1, D), bb1.reshape(1, D), g2.reshape(1, D),
      bb2.reshape(1, D))


def _readout(h, readout_ids, w_gen, b_gen):
    return pl.pallas_call(
        _readout_body,
        in_specs=[
            pl.BlockSpec(memory_space=pltpu.MemorySpace.VMEM),
            pl.BlockSpec(memory_space=pltpu.MemorySpace.SMEM),
            pl.BlockSpec(memory_space=pltpu.MemorySpace.VMEM),
            pl.BlockSpec(memory_space=pltpu.MemorySpace.VMEM),
        ],
        out_specs=pl.BlockSpec(memory_space=pltpu.MemorySpace.VMEM),
        out_shape=jax.ShapeDtypeStruct((NR, NC), jnp.float32),
        scratch_shapes=[pltpu.VMEM((NR, D), jnp.float32)],
        compiler_params=pltpu.CompilerParams(
            vmem_limit_bytes=100 * 2**20),
    )(h, readout_ids, w_gen, b_gen.reshape(1, NC))


def kernel(x, pos, edge_index, leaf_ids, readout_ids, emb, norm_g, norm_b,
           Wq, Wk, Wv, Wo, W1, b1, W2, b2, ln1_g, ln1_b, ln2_g, ln2_b,
           W_gen, b_gen):
    src = edge_index[0]
    dst = edge_index[1]
    x_pad = jnp.concatenate(
        [x, jnp.broadcast_to(x[:1], (_NLP - NL,))]).astype(jnp.int32)
    he_raw = _embed_gather(emb, x_pad)[:NL]
    posf = pos.astype(jnp.float32).reshape(NL, 1)
    he = _pe_ln(he_raw, posf, norm_g, norm_b)
    h = _scatter_leaves(he, leaf_ids)
    dst2 = dst.reshape(E, 1)
    for l in range(L):
        q3, k3, v3 = _qkv(h, Wq[l], Wk[l], Wv[l])
        pe = _edgeprod_sc(
            q3.reshape(_NSC * N, _HHALF),
            k3.reshape(_NSC * N, _HHALF),
            src, dst)
        aw, den = _score_tc(pe.reshape(_NSC, E, _HHALF), dst2)
        sc_out, den_t = _attn_sc(
            v3.reshape(_NSC * N, _HHALF),
            aw.reshape(_NSC * E, _HHALF),
            den, src, dst)
        den8 = den_t[:_ND].reshape(_ND * 16, H)[:N]
        h = _post(h, sc_out.reshape(_NSC, N, _HHALF), den8, Wo[l], W1[l],
                  b1[l], W2[l], b2[l], ln1_g[l], ln1_b[l], ln2_g[l],
                  ln2_b[l])
    return _readout(h, readout_ids, W_gen, b_gen)


# speedup vs baseline: 15.7313x; 1.1632x over previous
import functools

import jax
import jax.numpy as jnp
import numpy as np
from jax import lax
from jax.experimental import pallas as pl
from jax.experimental.pallas import tpu as pltpu
from jax.experimental.pallas import tpu_sc as plsc

N = 10000
E = 160000
NL = 5000
NR = 100
V = 10000
D = 256
DFF = 1024
H = 8
L = 3
NC = 128
DH = D // H

_NSC = 2
_NT = 16
_CH = 400
_CC = 160
_CT = 80
_CHE = 80
_EPT = E // _NT
_NCHUNK = _EPT // _CH
_HHALF = D // _NSC
_HPS = H // _NSC
_ND = 640
_RPT = N // _NT
_NLP = 5120
_INV = 1.0 / np.sqrt(DH)

_mesh = plsc.VectorSubcoreMesh(core_axis_name="c", subcore_axis_name="s")


@functools.partial(
    pl.kernel,
    mesh=_mesh,
    out_type=jax.ShapeDtypeStruct((_NLP, D), jnp.float32),
    scratch_types=[
        pltpu.VMEM((_CHE,), jnp.int32),
        pltpu.VMEM((_CHE, D), jnp.float32),
        pltpu.SemaphoreType.DMA,
    ],
)
def _embed_gather(emb_hbm, x_hbm, out_hbm, ib, rows, sem):
    c = lax.axis_index("c")
    s = lax.axis_index("s")
    w = s * _NSC + c
    for kk in range(_NLP // (32 * _CHE)):
        base = pl.multiple_of(w * (_NLP // 32) + kk * _CHE, 8)
        pltpu.sync_copy(x_hbm.at[pl.ds(base, _CHE)], ib)
        pltpu.async_copy(emb_hbm.at[ib], rows, sem).wait()
        pltpu.sync_copy(rows, out_hbm.at[pl.ds(base, _CHE)])


@functools.partial(
    pl.kernel,
    mesh=_mesh,
    out_type=jax.ShapeDtypeStruct((_NSC * E, _HHALF), jnp.float32),
    scratch_types=[
        pltpu.VMEM((_CH,), jnp.int32),
        pltpu.VMEM((_CH,), jnp.int32),
        pltpu.VMEM((_CH, _HHALF), jnp.float32),
        pltpu.VMEM((_CH, _HHALF), jnp.float32),
        pltpu.SemaphoreType.DMA,
    ],
)
def _edgeprod_sc(qc, kc, srcs, dsts, out, sbuf, dbuf, qd, ks, sem):
    c = lax.axis_index("c")
    s = lax.axis_index("s")
    cn = c * N

    def _chunk(ci, carry):
        start = pl.multiple_of(s * _EPT + ci * _CH, 8)
        pltpu.sync_copy(srcs.at[pl.ds(start, _CH)], sbuf)
        pltpu.sync_copy(dsts.at[pl.ds(start, _CH)], dbuf)
        for i in range(_CH // 16):
            sl = pl.ds(i * 16, 16)
            sbuf[sl] = sbuf[sl] + cn
            dbuf[sl] = dbuf[sl] + cn
        cp1 = pltpu.async_copy(qc.at[dbuf], qd, sem)
        cp2 = pltpu.async_copy(kc.at[sbuf], ks, sem)
        cp1.wait()
        cp2.wait()

        def _edge(e, carry2):
            for half in range(_HHALF // 16):
                sl = pl.ds(half * 16, 16)
                qd[e, sl] = qd[e, sl] * ks[e, sl]
            return carry2

        lax.fori_loop(0, _CH, _edge, 0)
        pltpu.sync_copy(qd, out.at[pl.ds(c * E + start, _CH)])
        return carry

    lax.fori_loop(0, _NCHUNK, _chunk, 0)


_BE = 2000


def _score_body(pe_ref, dst_ref, aw_ref, den_ref):
    pfull = jnp.concatenate([pe_ref[0], pe_ref[1]], axis=1)
    cc0 = lax.broadcasted_iota(jnp.int32, (D, H), 0)
    hh = lax.broadcasted_iota(jnp.int32, (D, H), 1)
    sel = ((cc0 // DH) == hh).astype(jnp.float32)
    a = jnp.exp(jnp.minimum(
        jnp.dot(pfull, sel, preferred_element_type=jnp.float32) * _INV,
        60.0))
    jj = lax.broadcasted_iota(jnp.int32, (_HPS, _HHALF), 0)
    col = lax.broadcasted_iota(jnp.int32, (_HPS, _HHALF), 1)
    rep = ((col // DH) == jj).astype(jnp.float32)
    aw_ref[0] = jnp.dot(a[:, :_HPS], rep, preferred_element_type=jnp.float32)
    aw_ref[1] = jnp.dot(a[:, _HPS:], rep, preferred_element_type=jnp.float32)
    a_rep = jnp.concatenate([a] * 16, axis=1)
    lanes = lax.broadcasted_iota(jnp.int32, (_BE, _HHALF), 1)
    m = (lanes // H) == (dst_ref[...] % 16)
    den_ref[...] = a_rep * m.astype(jnp.float32)


def _score_tc(pe, dst2):
    grid = (E // _BE,)
    return pl.pallas_call(
        _score_body,
        grid=grid,
        in_specs=[
            pl.BlockSpec((_NSC, _BE, _HHALF), lambda i: (0, i, 0)),
            pl.BlockSpec((_BE, 1), lambda i: (i, 0)),
        ],
        out_specs=(
            pl.BlockSpec((_NSC, _BE, _HHALF), lambda i: (0, i, 0)),
            pl.BlockSpec((_BE, _HHALF), lambda i: (i, 0)),
        ),
        out_shape=(
            jax.ShapeDtypeStruct((_NSC, E, _HHALF), jnp.float32),
            jax.ShapeDtypeStruct((E, _HHALF), jnp.float32),
        ),
    )(pe, dst2)


@functools.partial(
    pl.kernel,
    mesh=_mesh,
    out_type=(
        jax.ShapeDtypeStruct((_NSC * N, _HHALF), jnp.float32),
        jax.ShapeDtypeStruct((_NSC * _ND, _HHALF), jnp.float32),
    ),
    scratch_types=[
        pltpu.VMEM((_CC,), jnp.int32),
        pltpu.VMEM((_CC,), jnp.int32),
        pltpu.VMEM((_CC,), jnp.int32),
        pltpu.VMEM((_CT,), jnp.int32),
        pltpu.VMEM((_CT,), jnp.int32),
        pltpu.VMEM((_CT,), jnp.int32),
        pltpu.VMEM((_CC, _HHALF), jnp.float32),
        pltpu.VMEM((_CC, _HHALF), jnp.float32),
        pltpu.VMEM_SHARED((N, _HHALF), jnp.float32),
        pltpu.VMEM_SHARED((_ND, _HHALF), jnp.float32),
        pltpu.SemaphoreType.DMA,
    ],
)
def _attn_sc(vc, aw, den, srcs, dsts, out, dout, sbuf, dbuf, dbuf16,
             tsb, tdb, tdb16, vs, awb, shared, sharedd, sem):
    c = lax.axis_index("c")
    s = lax.axis_index("s")
    cn = c * N
    zero16 = jnp.zeros((16,), jnp.float32)

    def _zrow(r, carry):
        for c16 in range(_HHALF // 16):
            vs[r, pl.ds(c16 * 16, 16)] = zero16
        return carry

    lax.fori_loop(0, _CC, _zrow, 0)
    nchn = N // _CHE

    def _zchunk(k, carry):
        j = jnp.minimum(s + k * _NT, nchn - 1)
        base = pl.multiple_of(j * _CHE, 8)
        pltpu.sync_copy(vs.at[pl.ds(0, _CHE)], shared.at[pl.ds(base, _CHE)])
        return carry

    lax.fori_loop(0, (nchn + _NT - 1) // _NT, _zchunk, 0)
    jd = jnp.minimum(s, _ND // _CHE - 1)
    based = pl.multiple_of(jd * _CHE, 8)
    pltpu.sync_copy(vs.at[pl.ds(0, _CHE)], sharedd.at[pl.ds(based, _CHE)])
    plsc.subcore_barrier()

    def _do_chunk(start, sz, sb, db, db16):
        pltpu.sync_copy(srcs.at[pl.ds(start, sz)], sb)
        pltpu.sync_copy(dsts.at[pl.ds(start, sz)], db)
        for i in range(sz // 16):
            sl = pl.ds(i * 16, 16)
            db16[sl] = lax.shift_right_logical(db[sl], 4)
            sb[sl] = sb[sl] + cn
        vsl = vs.at[pl.ds(0, sz)]
        awl = awb.at[pl.ds(0, sz)]
        cp1 = pltpu.async_copy(vc.at[sb], vsl, sem)
        cp2 = pltpu.async_copy(aw.at[pl.ds(c * E + start, sz)], awl, sem)
        cp1.wait()
        cp2.wait()

        def _edge(e, carry2):
            for half in range(_HHALF // 16):
                sl = pl.ds(half * 16, 16)
                awb[e, sl] = awb[e, sl] * vs[e, sl]
            return carry2

        lax.fori_loop(0, sz, _edge, 0)
        pltpu.sync_copy(den.at[pl.ds(start, sz)], vsl)
        pltpu.sync_copy(awl, shared.at[db], add=True)
        pltpu.sync_copy(vsl, sharedd.at[db16], add=True)

    def _chunk(ci, carry):
        start = pl.multiple_of(s * _EPT + ci * _CC, 8)
        _do_chunk(start, _CC, sbuf, dbuf, dbuf16)
        return carry

    lax.fori_loop(0, _EPT // _CC, _chunk, 0)
    _do_chunk(pl.multiple_of(s * _EPT + (_EPT // _CC) * _CC, 8), _CT,
              tsb, tdb, tdb16)
    plsc.subcore_barrier()

    def _wchunk(k, carry):
        j = jnp.minimum(s + k * _NT, nchn - 1)
        base = pl.multiple_of(j * _CHE, 8)
        pltpu.sync_copy(shared.at[pl.ds(base, _CHE)],
                        out.at[pl.ds(cn + base, _CHE)])
        return carry

    lax.fori_loop(0, (nchn + _NT - 1) // _NT, _wchunk, 0)
    pltpu.sync_copy(sharedd.at[pl.ds(based, _CHE)],
                    dout.at[pl.ds(c * _ND + based, _CHE)])


_BN = 1000


def _ln(x, g, b):
    m = jnp.mean(x, axis=-1, keepdims=True)
    v = jnp.mean((x - m) ** 2, axis=-1, keepdims=True)
    return (x - m) / jnp.sqrt(v + 1e-5) * g + b


def _pe_ln_body(her_ref, posf_ref, g_ref, b_ref, out_ref):
    her = her_ref[...]
    posf = posf_ref[...]
    ci = lax.broadcasted_iota(jnp.int32, (her.shape[0], D), 1)
    i_f = (ci // 2).astype(jnp.float32)
    freq = jnp.exp((-np.log(10000.0) * 2.0 / D) * i_f)
    ang = posf * freq
    pe = jnp.where((ci % 2) == 0, jnp.sin(ang), jnp.cos(ang))
    out_ref[...] = _ln(her + pe, g_ref[...], b_ref[...])


def _scatter_body(he_ref, ids_ref, out_ref):
    out_ref[...] = jnp.zeros((N, D), jnp.float32)

    def body(i, carry):
        idv = ids_ref[i]
        out_ref[pl.ds(idv, 1), :] = he_ref[pl.ds(i, 1), :]
        return carry

    lax.fori_loop(0, NL, body, 0)


def _qkv_body(h_ref, wq_ref, wk_ref, wv_ref, q_ref, k_ref, v_ref):
    h = h_ref[...]
    for w_ref, o_ref in ((wq_ref, q_ref), (wk_ref, k_ref), (wv_ref, v_ref)):
        o = jnp.dot(h, w_ref[...], preferred_element_type=jnp.float32)
        o_ref[0] = o[:, :_HHALF]
        o_ref[1] = o[:, _HHALF:]


def _post_body(h_ref, sc_ref, den_ref, wo_ref, w1_ref, b1_ref, w2_ref,
               b2_ref, g1_ref, bb1_ref, g2_ref, bb2_ref, out_ref):
    h = h_ref[...]
    num = jnp.concatenate([sc_ref[0], sc_ref[1]], axis=1)
    den8 = den_ref[...] + 1e-9
    hh = lax.broadcasted_iota(jnp.int32, (H, D), 0)
    cc = lax.broadcasted_iota(jnp.int32, (H, D), 1)
    sel = ((cc // DH) == hh).astype(jnp.float32)
    den = jnp.dot(den8, sel, preferred_element_type=jnp.float32)
    agg = num / den
    o = jnp.dot(agg, wo_ref[...], preferred_element_type=jnp.float32)
    h1 = _ln(h + o, g1_ref[...], bb1_ref[...])
    f = jnp.maximum(
        jnp.dot(h1, w1_ref[...], preferred_element_type=jnp.float32)
        + b1_ref[...], 0.0)
    f2 = jnp.dot(f, w2_ref[...], preferred_element_type=jnp.float32) \
        + b2_ref[...]
    out_ref[...] = _ln(h1 + f2, g2_ref[...], bb2_ref[...])


def _readout_body(h_ref, ids_ref, wg_ref, bg_ref, out_ref, rows_ref):
    def body(j, carry):
        rid = ids_ref[j]
        rows_ref[pl.ds(j, 1), :] = h_ref[pl.ds(rid, 1), :]
        return carry

    lax.fori_loop(0, NR, body, 0)
    xs = rows_ref[...]
    z = jnp.dot(xs, wg_ref[...], preferred_element_type=jnp.float32) \
        + bg_ref[...]
    m = jnp.max(z, axis=-1, keepdims=True)
    out_ref[...] = z - m - jnp.log(
        jnp.sum(jnp.exp(z - m), axis=-1, keepdims=True))


def _full_spec(shape):
    return pl.BlockSpec(shape, lambda *_: (0,) * len(shape))


def _pe_ln(he_raw, posf, norm_g, norm_b):
    grid = (NL // _BN,)
    return pl.pallas_call(
        _pe_ln_body,
        grid=grid,
        in_specs=[
            pl.BlockSpec((_BN, D), lambda i: (i, 0)),
            pl.BlockSpec((_BN, 1), lambda i: (i, 0)),
            _full_spec((1, D)),
            _full_spec((1, D)),
        ],
        out_specs=pl.BlockSpec((_BN, D), lambda i: (i, 0)),
        out_shape=jax.ShapeDtypeStruct((NL, D), jnp.float32),
    )(he_raw, posf, norm_g.reshape(1, D), norm_b.reshape(1, D))


def _scatter_leaves(he, leaf_ids):
    return pl.pallas_call(
        _scatter_body,
        in_specs=[
            pl.BlockSpec(memory_space=pltpu.MemorySpace.VMEM),
            pl.BlockSpec(memory_space=pltpu.MemorySpace.SMEM),
        ],
        out_specs=pl.BlockSpec(memory_space=pltpu.MemorySpace.VMEM),
        out_shape=jax.ShapeDtypeStruct((N, D), jnp.float32),
        compiler_params=pltpu.CompilerParams(
            vmem_limit_bytes=100 * 2**20),
    )(he, leaf_ids)


def _qkv(h, wq, wk, wv):
    grid = (N // _BN,)
    o_spec = pl.BlockSpec((_NSC, _BN, _HHALF), lambda i: (0, i, 0))
    o_shape = jax.ShapeDtypeStruct((_NSC, N, _HHALF), jnp.float32)
    return pl.pallas_call(
        _qkv_body,
        grid=grid,
        in_specs=[
            pl.BlockSpec((_BN, D), lambda i: (i, 0)),
            _full_spec((D, D)),
            _full_spec((D, D)),
            _full_spec((D, D)),
        ],
        out_specs=(o_spec, o_spec, o_spec),
        out_shape=(o_shape, o_shape, o_shape),
    )(h, wq, wk, wv)


def _post(h, sc_out, den8, wo, w1, b1, w2, b2, g1, bb1, g2, bb2):
    grid = (N // _BN,)
    return pl.pallas_call(
        _post_body,
        grid=grid,
        in_specs=[
            pl.BlockSpec((_BN, D), lambda i: (i, 0)),
            pl.BlockSpec((_NSC, _BN, _HHALF), lambda i: (0, i, 0)),
            pl.BlockSpec((_BN, H), lambda i: (i, 0)),
            _full_spec((D, D)),
            _full_spec((D, DFF)),
            _full_spec((1, DFF)),
            _full_spec((DFF, D)),
            _full_spec((1, D)),
            _full_spec((1, D)),
            _full_spec((1, D)),
            _full_spec((1, D)),
            _full_spec((1, D)),
        ],
        out_specs=pl.BlockSpec((_BN, D), lambda i: (i, 0)),
        out_shape=jax.ShapeDtypeStruct((N, D), jnp.float32),
    )(h, sc_out, den8, wo, w1, b1.reshape(1, DFF), w2, b2.reshape(1, D),
      g1.reshape(1, D), bb1.reshape(1, D), g2.reshape(1, D),
      bb2.reshape(1, D))


def _readout(h, readout_ids, w_gen, b_gen):
    return pl.pallas_call(
        _readout_body,
        in_specs=[
            pl.BlockSpec(memory_space=pltpu.MemorySpace.VMEM),
            pl.BlockSpec(memory_space=pltpu.MemorySpace.SMEM),
            pl.BlockSpec(memory_space=pltpu.MemorySpace.VMEM),
            pl.BlockSpec(memory_space=pltpu.MemorySpace.VMEM),
        ],
        out_specs=pl.BlockSpec(memory_space=pltpu.MemorySpace.VMEM),
        out_shape=jax.ShapeDtypeStruct((NR, NC), jnp.float32),
        scratch_shapes=[pltpu.VMEM((NR, D), jnp.float32)],
        compiler_params=pltpu.CompilerParams(
            vmem_limit_bytes=100 * 2**20),
    )(h, readout_ids, w_gen, b_gen.reshape(1, NC))


def kernel(x, pos, edge_index, leaf_ids, readout_ids, emb, norm_g, norm_b,
           Wq, Wk, Wv, Wo, W1, b1, W2, b2, ln1_g, ln1_b, ln2_g, ln2_b,
           W_gen, b_gen):
    src = edge_index[0]
    dst = edge_index[1]
    x_pad = jnp.concatenate(
        [x, jnp.broadcast_to(x[:1], (_NLP - NL,))]).astype(jnp.int32)
    he_raw = _embed_gather(emb, x_pad)[:NL]
    posf = pos.astype(jnp.float32).reshape(NL, 1)
    he = _pe_ln(he_raw, posf, norm_g, norm_b)
    h = _scatter_leaves(he, leaf_ids)
    dst2 = dst.reshape(E, 1)
    for l in range(L):
        q3, k3, v3 = _qkv(h, Wq[l], Wk[l], Wv[l])
        pe = _edgeprod_sc(
            q3.reshape(_NSC * N, _HHALF),
            k3.reshape(_NSC * N, _HHALF),
            src, dst)
        aw, den = _score_tc(pe.reshape(_NSC, E, _HHALF), dst2)
        sc_out, den_t = _attn_sc(
            v3.reshape(_NSC * N, _HHALF),
            aw.reshape(_NSC * E, _HHALF),
            den, src, dst)
        den8 = den_t[:_ND].reshape(_ND * 16, H)[:N]
        h = _post(h, sc_out.reshape(_NSC, N, _HHALF), den8, Wo[l], W1[l],
                  b1[l], W2[l], b2[l], ln1_g[l], ln1_b[l], ln2_g[l],
                  ln2_b[l])
    return _readout(h, readout_ids, W_gen, b_gen)
